# baseline (speedup 1.0000x reference)
import functools

import jax
import jax.numpy as jnp
from jax import lax
from jax.experimental import pallas as pl
from jax.experimental.pallas import tpu as pltpu
from jax.experimental.pallas import tpu_sc as plsc

_NC = 2
_NS = 16


def kernel(input_, weight):
    B, H = input_.shape
    V, D = weight.shape
    BH = B // 4
    NU = 4 * H
    n_u = (NU + _NS - 1) // _NS
    d_per_c = D // _NC
    idx2 = input_.T.reshape(NU, BH)
    wT = weight.T

    mesh = plsc.VectorSubcoreMesh(core_axis_name="c", subcore_axis_name="s")

    @functools.partial(
        pl.kernel,
        out_type=jax.ShapeDtypeStruct((H * D, B), jnp.float32),
        mesh=mesh,
        scratch_types=[
            pltpu.VMEM_SHARED((V,), jnp.float32),
            [pltpu.VMEM((BH,), jnp.int32) for _ in range(n_u)],
            [pltpu.VMEM((BH,), jnp.float32) for _ in range(2)],
            [pltpu.SemaphoreType.DMA for _ in range(2)],
            [pltpu.SemaphoreType.DMA for _ in range(2)],
        ],
    )
    def emb(idx_hbm, wT_hbm, out_hbm, col, idx_v, dst_v, gsem, wsem):
        def gat(k):
            return pltpu.make_async_copy(col.at[idx_v[k]], dst_v[k % 2], gsem[k % 2])

        c = lax.axis_index("c")
        s = lax.axis_index("s")
        d0 = c * d_per_c

        for k in range(n_u):
            u = s + k * _NS
            @pl.when(u < NU)
            def _load(k=k, u=u):
                pltpu.sync_copy(idx_hbm.at[u], idx_v[k])

        @pl.loop(0, d_per_c)
        def _body(j):
            @pl.when(s == 0)
            def _stage():
                pltpu.sync_copy(wT_hbm.at[d0 + j], col)
            plsc.subcore_barrier()

            def wrt(k, jj):
                u = s + k * _NS
                h = u // 4
                bh = u % 4
                boff = pl.multiple_of(bh * BH, BH)
                return pltpu.make_async_copy(
                    dst_v[k % 2], out_hbm.at[h * D + d0 + jj, pl.ds(boff, BH)],
                    wsem[k % 2])

            gat(0).start()
            for k in range(n_u):
                u = s + k * _NS
                if k + 1 < n_u:
                    if k >= 1:
                        up = s + (k - 1) * _NS
                        @pl.when(up < NU)
                        def _wprev(k=k):
                            wrt(k - 1, j).wait()
                    un = s + (k + 1) * _NS
                    @pl.when(un < NU)
                    def _nxt(k=k):
                        gat(k + 1).start()
                @pl.when(u < NU)
                def _one(k=k, u=u):
                    gat(k).wait()
                    wrt(k, j).start()
            for k in (n_u - 2, n_u - 1):
                u = s + k * _NS
                @pl.when(u < NU)
                def _wdrain(k=k):
                    wrt(k, j).wait()
            plsc.subcore_barrier()

    out2 = emb(idx2, wT)
    return out2.reshape(H, D, B).transpose(2, 0, 1)


# speedup vs baseline: 1.0015x; 1.0015x over previous
import functools

import jax
import jax.numpy as jnp
from jax import lax
from jax.experimental import pallas as pl
from jax.experimental.pallas import tpu as pltpu
from jax.experimental.pallas import tpu_sc as plsc

_NC = 2
_NS = 16


def kernel(input_, weight):
    B, H = input_.shape
    V, D = weight.shape
    BH = B // 4
    NU = 4 * H
    n_u = (NU + _NS - 1) // _NS
    d_per_c = D // _NC
    idx2 = input_.T.reshape(NU, BH)
    wT = weight.T

    mesh = plsc.VectorSubcoreMesh(core_axis_name="c", subcore_axis_name="s")

    @functools.partial(
        pl.kernel,
        out_type=jax.ShapeDtypeStruct((H * D, B), jnp.float32),
        mesh=mesh,
        scratch_types=[
            pltpu.VMEM_SHARED((V,), jnp.float32),
            [pltpu.VMEM((BH,), jnp.int32) for _ in range(n_u)],
            [pltpu.VMEM((BH,), jnp.float32) for _ in range(2)],
            [pltpu.SemaphoreType.DMA for _ in range(2)],
        ],
    )
    def emb(idx_hbm, wT_hbm, out_hbm, col, idx_v, dst_v, gsem):
        def gat(k):
            return pltpu.make_async_copy(col.at[idx_v[k]], dst_v[k % 2], gsem[k % 2])

        c = lax.axis_index("c")
        s = lax.axis_index("s")
        d0 = c * d_per_c

        for k in range(n_u):
            u = s + k * _NS
            @pl.when(u < NU)
            def _load(k=k, u=u):
                pltpu.sync_copy(idx_hbm.at[u], idx_v[k])

        @pl.loop(0, d_per_c)
        def _body(j):
            @pl.when(s == 0)
            def _stage():
                pltpu.sync_copy(wT_hbm.at[d0 + j], col)
            plsc.subcore_barrier()
            gat(0).start()
            for k in range(n_u):
                u = s + k * _NS
                if k + 1 < n_u:
                    un = s + (k + 1) * _NS
                    @pl.when(un < NU)
                    def _nxt(k=k):
                        gat(k + 1).start()
                @pl.when(u < NU)
                def _one(k=k, u=u):
                    h = u // 4
                    bh = u % 4
                    gat(k).wait()
                    boff = pl.multiple_of(bh * BH, BH)
                    pltpu.sync_copy(dst_v[k % 2], out_hbm.at[h * D + d0 + j, pl.ds(boff, BH)])
            plsc.subcore_barrier()

    out2 = emb(idx2, wT)
    return out2.reshape(H, D, B).transpose(2, 0, 1)
